# Initial kernel scaffold; baseline (speedup 1.0000x reference)
import jax
import jax.numpy as jnp
from jax import lax
from jax.experimental import pallas as pl
from jax.experimental.pallas import tpu as pltpu
from jax.experimental.pallas import tpu_sc as plsc

N = 10000
E = 320000
NC = 2
NS = 16
C = 80
RPS = N // NS

_F32 = jnp.float32


def _mesh():
    return plsc.VectorSubcoreMesh(
        core_axis_name="c", subcore_axis_name="s", num_cores=NC, num_subcores=NS
    )


def _seg_a_body(x_hbm, src_hbm, dst_hbm, z128_hbm, z16_hbm, ones_hbm,
                p_hbm, dg_hbm,
                sidx, didx, rows, ones_v, acc, dacc, sem):
    c = lax.axis_index("c")
    s = lax.axis_index("s")
    row0 = s * RPS
    pltpu.sync_copy(z128_hbm.at[pl.ds(row0, RPS)], acc.at[pl.ds(row0, RPS)])
    pltpu.sync_copy(z16_hbm.at[pl.ds(row0, RPS)], dacc.at[pl.ds(row0, RPS)])
    pltpu.sync_copy(ones_hbm, ones_v)
    plsc.subcore_barrier()

    epw = E // (NC * NS)
    e0 = (c * NS + s) * epw

    def chunk(i, carry):
        base = e0 + i * C
        pltpu.sync_copy(src_hbm.at[pl.ds(base, C)], sidx)
        pltpu.async_copy(x_hbm.at[sidx], rows, sem).wait()
        pltpu.sync_copy(dst_hbm.at[pl.ds(base, C)], didx)
        pltpu.sync_copy(rows, acc.at[didx], add=True)
        pltpu.sync_copy(ones_v, dacc.at[didx], add=True)
        return carry

    lax.fori_loop(0, epw // C, chunk, 0)
    plsc.subcore_barrier()
    pltpu.sync_copy(acc.at[pl.ds(row0, RPS)], p_hbm.at[c, pl.ds(row0, RPS)])
    pltpu.sync_copy(dacc.at[pl.ds(row0, RPS)], dg_hbm.at[c, pl.ds(row0, RPS)])


_seg_a = pl.kernel(
    _seg_a_body,
    out_type=[jax.ShapeDtypeStruct((NC, N, 128), _F32),
              jax.ShapeDtypeStruct((NC, N, 16), _F32)],
    mesh=_mesh(),
    scratch_types=[pltpu.VMEM((C,), jnp.int32), pltpu.VMEM((C,), jnp.int32),
                   pltpu.VMEM((C, 128), _F32), pltpu.VMEM((C, 16), _F32),
                   pltpu.VMEM_SHARED((N, 128), _F32),
                   pltpu.VMEM_SHARED((N, 16), _F32),
                   pltpu.SemaphoreType.DMA],
)


def _seg_b_body(t_hbm, src_hbm, dst_hbm, z128_hbm,
                agg_hbm,
                sidx, didx, rows, acc, sem):
    c = lax.axis_index("c")
    s = lax.axis_index("s")
    row0 = s * RPS
    pltpu.sync_copy(z128_hbm.at[pl.ds(row0, RPS)], acc.at[pl.ds(row0, RPS)])
    plsc.subcore_barrier()

    eps = E // NS
    e0 = s * eps
    off = c * N

    def chunk(i, carry):
        base = e0 + i * C
        pltpu.sync_copy(src_hbm.at[pl.ds(base, C)], sidx)
        for k in range(C // 16):
            sidx[pl.ds(k * 16, 16)] = sidx[pl.ds(k * 16, 16)] + off
        pltpu.async_copy(t_hbm.at[sidx], rows, sem).wait()
        pltpu.sync_copy(dst_hbm.at[pl.ds(base, C)], didx)
        pltpu.sync_copy(rows, acc.at[didx], add=True)
        return carry

    lax.fori_loop(0, eps // C, chunk, 0)
    plsc.subcore_barrier()
    pltpu.sync_copy(acc.at[pl.ds(row0, RPS)], agg_hbm.at[c, pl.ds(row0, RPS)])


_seg_b = pl.kernel(
    _seg_b_body,
    out_type=[jax.ShapeDtypeStruct((NC, N, 128), _F32)],
    mesh=_mesh(),
    scratch_types=[pltpu.VMEM((C,), jnp.int32), pltpu.VMEM((C,), jnp.int32),
                   pltpu.VMEM((C, 128), _F32),
                   pltpu.VMEM_SHARED((N, 128), _F32),
                   pltpu.SemaphoreType.DMA],
)


def _seg_c_body(t_hbm, src_hbm, dst_hbm, z128_hbm,
                p_hbm,
                sidx, didx, rows, acc, sem):
    c = lax.axis_index("c")
    s = lax.axis_index("s")
    row0 = s * RPS
    pltpu.sync_copy(z128_hbm.at[pl.ds(row0, RPS)], acc.at[pl.ds(row0, RPS)])
    plsc.subcore_barrier()

    epw = E // (NC * NS)
    e0 = (c * NS + s) * epw

    def chunk(i, carry):
        base = e0 + i * C
        pltpu.sync_copy(src_hbm.at[pl.ds(base, C)], sidx)
        pltpu.async_copy(t_hbm.at[sidx], rows, sem).wait()
        pltpu.sync_copy(dst_hbm.at[pl.ds(base, C)], didx)
        pltpu.sync_copy(rows, acc.at[didx], add=True)
        return carry

    lax.fori_loop(0, epw // C, chunk, 0)
    plsc.subcore_barrier()
    pltpu.sync_copy(acc.at[pl.ds(row0, RPS)], p_hbm.at[c, pl.ds(row0, RPS)])


_seg_c = pl.kernel(
    _seg_c_body,
    out_type=[jax.ShapeDtypeStruct((NC, N, 128), _F32)],
    mesh=_mesh(),
    scratch_types=[pltpu.VMEM((C,), jnp.int32), pltpu.VMEM((C,), jnp.int32),
                   pltpu.VMEM((C, 128), _F32),
                   pltpu.VMEM_SHARED((N, 128), _F32),
                   pltpu.SemaphoreType.DMA],
)


BN = 2000

_HI = lax.Precision.HIGHEST


def _tc1_body(p_ref, dg_ref, w_ref, b_ref, h_ref, inv_ref):
    agg = p_ref[0] + p_ref[1]
    deg = dg_ref[0][:, 0:1] + dg_ref[1][:, 0:1]
    inv = 1.0 / jnp.maximum(deg, 1.0)
    inv_ref[...] = jnp.broadcast_to(inv, (BN, 16))
    h = jnp.dot(agg * inv, w_ref[...], preferred_element_type=_F32, precision=_HI)
    h = jnp.maximum(h + b_ref[...] + 0.01, 0.0)
    h_ref[0] = h[:, :128]
    h_ref[1] = h[:, 128:]


_tc1 = pl.pallas_call(
    _tc1_body,
    grid=(N // BN,),
    in_specs=[pl.BlockSpec((NC, BN, 128), lambda i: (0, i, 0)),
              pl.BlockSpec((NC, BN, 16), lambda i: (0, i, 0)),
              pl.BlockSpec((128, 256), lambda i: (0, 0)),
              pl.BlockSpec((1, 256), lambda i: (0, 0))],
    out_specs=[pl.BlockSpec((NC, BN, 128), lambda i: (0, i, 0)),
               pl.BlockSpec((BN, 16), lambda i: (i, 0))],
    out_shape=[jax.ShapeDtypeStruct((NC, N, 128), _F32),
               jax.ShapeDtypeStruct((N, 16), _F32)],
)


def _tc2_body(a_ref, inv_ref, w2_ref, b2_ref, w3_ref, t_ref):
    inv = inv_ref[:, 0:1]
    a0 = a_ref[0] * inv
    a1 = a_ref[1] * inv
    h = (jnp.dot(a0, w2_ref[0:128], preferred_element_type=_F32, precision=_HI)
         + jnp.dot(a1, w2_ref[128:256], preferred_element_type=_F32, precision=_HI)
         + b2_ref[...] + 0.01)
    h = jnp.maximum(h, 0.0)
    t_ref[...] = jnp.dot(h, w3_ref[...], preferred_element_type=_F32, precision=_HI)


_tc2 = pl.pallas_call(
    _tc2_body,
    grid=(N // BN,),
    in_specs=[pl.BlockSpec((NC, BN, 128), lambda i: (0, i, 0)),
              pl.BlockSpec((BN, 16), lambda i: (i, 0)),
              pl.BlockSpec((256, 256), lambda i: (0, 0)),
              pl.BlockSpec((1, 256), lambda i: (0, 0)),
              pl.BlockSpec((256, 128), lambda i: (0, 0))],
    out_specs=pl.BlockSpec((BN, 128), lambda i: (i, 0)),
    out_shape=jax.ShapeDtypeStruct((N, 128), _F32),
)


def _tc3_body(p_ref, inv_ref, b_ref, o_ref):
    o_ref[...] = (p_ref[0] + p_ref[1]) * inv_ref[:, 0:1] + b_ref[...] + 0.01


_tc3 = pl.pallas_call(
    _tc3_body,
    grid=(N // BN,),
    in_specs=[pl.BlockSpec((NC, BN, 128), lambda i: (0, i, 0)),
              pl.BlockSpec((BN, 16), lambda i: (i, 0)),
              pl.BlockSpec((1, 128), lambda i: (0, 0))],
    out_specs=pl.BlockSpec((BN, 128), lambda i: (i, 0)),
    out_shape=jax.ShapeDtypeStruct((N, 128), _F32),
)


def kernel(x, edge_index, W1, b1, W2, b2, W3, b3):
    src = edge_index[0].astype(jnp.int32)
    dst = edge_index[1].astype(jnp.int32)
    z128 = jnp.zeros((N, 128), _F32)
    z16 = jnp.zeros((N, 16), _F32)
    ones = jnp.ones((C, 16), _F32)

    p1, degp = _seg_a(x, src, dst, z128, z16, ones)
    h1s, invd = _tc1(p1, degp, W1, b1.reshape(1, 256))
    agg2 = _seg_b(h1s.reshape(2 * N, 128), src, dst, z128)[0]
    t3 = _tc2(agg2, invd, W2, b2.reshape(1, 256), W3)
    p3 = _seg_c(t3, src, dst, z128)[0]
    return _tc3(p3, invd, b3.reshape(1, 128))


# speedup vs baseline: 2.4573x; 2.4573x over previous
import jax
import jax.numpy as jnp
from jax import lax
from jax.experimental import pallas as pl
from jax.experimental.pallas import tpu as pltpu
from jax.experimental.pallas import tpu_sc as plsc

N = 10000
NP = 10112
E = 320000
NC = 2
NS = 16
C = 80
RPS = NP // NS

_F32 = jnp.float32


def _mesh():
    return plsc.VectorSubcoreMesh(
        core_axis_name="c", subcore_axis_name="s", num_cores=NC, num_subcores=NS
    )


NP8 = 1280
RPS8 = NP8 // NS


def _seg_a_body(x_hbm, src_hbm, dst_hbm, z128_hbm, pat_hbm,
                p_hbm, dg_hbm,
                sidx, didx, d8, m8, rows, prow, acc, dacc, sem, psem):
    c = lax.axis_index("c")
    s = lax.axis_index("s")
    row0 = s * RPS
    dr0 = s * RPS8
    pltpu.sync_copy(z128_hbm.at[pl.ds(row0, RPS)], acc.at[pl.ds(row0, RPS)])
    pltpu.sync_copy(z128_hbm.at[pl.ds(dr0, RPS8)], dacc.at[pl.ds(dr0, RPS8)])
    plsc.subcore_barrier()

    epw = E // (NC * NS)
    e0 = (c * NS + s) * epw

    def chunk(i, carry):
        base = e0 + i * C
        pltpu.sync_copy(src_hbm.at[pl.ds(base, C)], sidx)
        cp = pltpu.async_copy(x_hbm.at[sidx], rows, sem)
        pltpu.sync_copy(dst_hbm.at[pl.ds(base, C)], didx)
        for k in range(C // 16):
            dv = didx[pl.ds(k * 16, 16)]
            d8[pl.ds(k * 16, 16)] = lax.shift_right_logical(dv, 3)
            m8[pl.ds(k * 16, 16)] = lax.bitwise_and(dv, 7)
        cq = pltpu.async_copy(pat_hbm.at[m8], prow, psem)
        cp.wait()
        pltpu.sync_copy(rows, acc.at[didx], add=True)
        cq.wait()
        pltpu.sync_copy(prow, dacc.at[d8], add=True)
        return carry

    lax.fori_loop(0, epw // C, chunk, 0)
    plsc.subcore_barrier()
    pltpu.sync_copy(acc.at[pl.ds(row0, RPS)], p_hbm.at[c, pl.ds(row0, RPS)])
    pltpu.sync_copy(dacc.at[pl.ds(dr0, RPS8)], dg_hbm.at[c, pl.ds(dr0, RPS8)])


_seg_a = pl.kernel(
    _seg_a_body,
    out_type=[jax.ShapeDtypeStruct((NC, NP, 128), _F32),
              jax.ShapeDtypeStruct((NC, NP8, 128), _F32)],
    mesh=_mesh(),
    scratch_types=[pltpu.VMEM((C,), jnp.int32), pltpu.VMEM((C,), jnp.int32),
                   pltpu.VMEM((C,), jnp.int32), pltpu.VMEM((C,), jnp.int32),
                   pltpu.VMEM((C, 128), _F32), pltpu.VMEM((C, 128), _F32),
                   pltpu.VMEM_SHARED((NP, 128), _F32),
                   pltpu.VMEM_SHARED((NP8, 128), _F32),
                   pltpu.SemaphoreType.DMA, pltpu.SemaphoreType.DMA],
)


def _seg_b_body(t_hbm, src_hbm, dst_hbm, z128_hbm,
                agg_hbm,
                sidx, didx, rows, acc, sem):
    c = lax.axis_index("c")
    s = lax.axis_index("s")
    row0 = s * RPS
    pltpu.sync_copy(z128_hbm.at[pl.ds(row0, RPS)], acc.at[pl.ds(row0, RPS)])
    plsc.subcore_barrier()

    eps = E // NS
    e0 = s * eps
    off = c * N

    def chunk(i, carry):
        base = e0 + i * C
        pltpu.sync_copy(src_hbm.at[pl.ds(base, C)], sidx)
        for k in range(C // 16):
            sidx[pl.ds(k * 16, 16)] = sidx[pl.ds(k * 16, 16)] + off
        pltpu.async_copy(t_hbm.at[sidx], rows, sem).wait()
        pltpu.sync_copy(dst_hbm.at[pl.ds(base, C)], didx)
        pltpu.sync_copy(rows, acc.at[didx], add=True)
        return carry

    lax.fori_loop(0, eps // C, chunk, 0)
    plsc.subcore_barrier()
    pltpu.sync_copy(acc.at[pl.ds(row0, RPS)], agg_hbm.at[c, pl.ds(row0, RPS)])


_seg_b = pl.kernel(
    _seg_b_body,
    out_type=[jax.ShapeDtypeStruct((NC, NP, 128), _F32)],
    mesh=_mesh(),
    scratch_types=[pltpu.VMEM((C,), jnp.int32), pltpu.VMEM((C,), jnp.int32),
                   pltpu.VMEM((C, 128), _F32),
                   pltpu.VMEM_SHARED((NP, 128), _F32),
                   pltpu.SemaphoreType.DMA],
)


def _seg_c_body(t_hbm, src_hbm, dst_hbm, z128_hbm,
                p_hbm,
                sidx, didx, rows, acc, sem):
    c = lax.axis_index("c")
    s = lax.axis_index("s")
    row0 = s * RPS
    pltpu.sync_copy(z128_hbm.at[pl.ds(row0, RPS)], acc.at[pl.ds(row0, RPS)])
    plsc.subcore_barrier()

    epw = E // (NC * NS)
    e0 = (c * NS + s) * epw

    def chunk(i, carry):
        base = e0 + i * C
        pltpu.sync_copy(src_hbm.at[pl.ds(base, C)], sidx)
        pltpu.async_copy(t_hbm.at[sidx], rows, sem).wait()
        pltpu.sync_copy(dst_hbm.at[pl.ds(base, C)], didx)
        pltpu.sync_copy(rows, acc.at[didx], add=True)
        return carry

    lax.fori_loop(0, epw // C, chunk, 0)
    plsc.subcore_barrier()
    pltpu.sync_copy(acc.at[pl.ds(row0, RPS)], p_hbm.at[c, pl.ds(row0, RPS)])


_seg_c = pl.kernel(
    _seg_c_body,
    out_type=[jax.ShapeDtypeStruct((NC, NP, 128), _F32)],
    mesh=_mesh(),
    scratch_types=[pltpu.VMEM((C,), jnp.int32), pltpu.VMEM((C,), jnp.int32),
                   pltpu.VMEM((C, 128), _F32),
                   pltpu.VMEM_SHARED((NP, 128), _F32),
                   pltpu.SemaphoreType.DMA],
)


BN = 2000

_HI = lax.Precision.HIGHEST


def _tc1_body(p_ref, d0_ref, d1_ref, w_ref, b_ref, h_ref, inv_ref):
    agg = p_ref[0] + p_ref[1]
    deg = d0_ref[:, 0:1] + d1_ref[:, 0:1]
    inv = 1.0 / jnp.maximum(deg, 1.0)
    inv_ref[...] = jnp.broadcast_to(inv, (BN, 16))
    h = jnp.dot(agg * inv, w_ref[...], preferred_element_type=_F32, precision=_HI)
    h = jnp.maximum(h + b_ref[...] + 0.01, 0.0)
    h_ref[0] = h[:, :128]
    h_ref[1] = h[:, 128:]


_tc1 = pl.pallas_call(
    _tc1_body,
    grid=(N // BN,),
    in_specs=[pl.BlockSpec((NC, BN, 128), lambda i: (0, i, 0)),
              pl.BlockSpec((BN, 16), lambda i: (i, 0)),
              pl.BlockSpec((BN, 16), lambda i: (i, 0)),
              pl.BlockSpec((128, 256), lambda i: (0, 0)),
              pl.BlockSpec((1, 256), lambda i: (0, 0))],
    out_specs=[pl.BlockSpec((NC, BN, 128), lambda i: (0, i, 0)),
               pl.BlockSpec((BN, 16), lambda i: (i, 0))],
    out_shape=[jax.ShapeDtypeStruct((NC, N, 128), _F32),
               jax.ShapeDtypeStruct((N, 16), _F32)],
)


def _tc2_body(a_ref, inv_ref, w2_ref, b2_ref, w3_ref, t_ref):
    inv = inv_ref[:, 0:1]
    a0 = a_ref[0] * inv
    a1 = a_ref[1] * inv
    h = (jnp.dot(a0, w2_ref[0:128], preferred_element_type=_F32, precision=_HI)
         + jnp.dot(a1, w2_ref[128:256], preferred_element_type=_F32, precision=_HI)
         + b2_ref[...] + 0.01)
    h = jnp.maximum(h, 0.0)
    t_ref[...] = jnp.dot(h, w3_ref[...], preferred_element_type=_F32, precision=_HI)


_tc2 = pl.pallas_call(
    _tc2_body,
    grid=(N // BN,),
    in_specs=[pl.BlockSpec((NC, BN, 128), lambda i: (0, i, 0)),
              pl.BlockSpec((BN, 16), lambda i: (i, 0)),
              pl.BlockSpec((256, 256), lambda i: (0, 0)),
              pl.BlockSpec((1, 256), lambda i: (0, 0)),
              pl.BlockSpec((256, 128), lambda i: (0, 0))],
    out_specs=pl.BlockSpec((BN, 128), lambda i: (i, 0)),
    out_shape=jax.ShapeDtypeStruct((N, 128), _F32),
)


def _tc3_body(p_ref, inv_ref, b_ref, o_ref):
    o_ref[...] = (p_ref[0] + p_ref[1]) * inv_ref[:, 0:1] + b_ref[...] + 0.01


_tc3 = pl.pallas_call(
    _tc3_body,
    grid=(N // BN,),
    in_specs=[pl.BlockSpec((NC, BN, 128), lambda i: (0, i, 0)),
              pl.BlockSpec((BN, 16), lambda i: (i, 0)),
              pl.BlockSpec((1, 128), lambda i: (0, 0))],
    out_specs=pl.BlockSpec((BN, 128), lambda i: (i, 0)),
    out_shape=jax.ShapeDtypeStruct((N, 128), _F32),
)


def kernel(x, edge_index, W1, b1, W2, b2, W3, b3):
    src = edge_index[0].astype(jnp.int32)
    dst = edge_index[1].astype(jnp.int32)
    z128 = jnp.zeros((NP, 128), _F32)
    pat = jnp.kron(jnp.eye(8, dtype=_F32), jnp.ones((1, 16), _F32))

    p1, degp = _seg_a(x, src, dst, z128, pat)
    d0 = degp[0].reshape(NP8 * 8, 16)[:N]
    d1 = degp[1].reshape(NP8 * 8, 16)[:N]
    h1s, invd = _tc1(p1, d0, d1, W1, b1.reshape(1, 256))
    agg2 = _seg_b(h1s.reshape(2 * N, 128), src, dst, z128)[0]
    t3 = _tc2(agg2, invd, W2, b2.reshape(1, 256), W3)
    p3 = _seg_c(t3, src, dst, z128)[0]
    return _tc3(p3, invd, b3.reshape(1, 128))
